# baseline (speedup 1.0000x reference)
import functools

import jax
import jax.numpy as jnp
from jax import lax
from jax.experimental import pallas as pl
from jax.experimental.pallas import tpu as pltpu
from jax.experimental.pallas import tpu_sc as plsc

_NCHUNK = 8
_CHUNK = 256


def _routing_kernel(x_ref, w_ref, e1_ref, e2_ref, g1_ref, g2_ref,
                    b1_ref, b2_ref, aux_ref, *, S, E, C):
    x = x_ref[0]
    w = w_ref[...]
    logits = jnp.dot(x, w, preferred_element_type=jnp.float32)

    m = jnp.max(logits, axis=-1, keepdims=True)
    ex = jnp.exp(logits - m)
    raw = ex / jnp.sum(ex, axis=-1, keepdims=True)

    iota_e = lax.broadcasted_iota(jnp.int32, (S, E), 1).astype(jnp.float32)

    mx1 = jnp.max(raw, axis=-1, keepdims=True)
    e1 = jnp.min(jnp.where(raw == mx1, iota_e, jnp.float32(E)),
                 axis=-1, keepdims=True)
    oh1 = (iota_e == e1).astype(jnp.float32)

    raw2 = raw * (1.0 - oh1)
    mx2 = jnp.max(raw2, axis=-1, keepdims=True)
    e2 = jnp.min(jnp.where(raw2 == mx2, iota_e, jnp.float32(E)),
                 axis=-1, keepdims=True)
    oh2 = (iota_e == e2).astype(jnp.float32)

    iota_row = lax.broadcasted_iota(jnp.int32, (_NCHUNK, S), 0)
    iota_col = lax.broadcasted_iota(jnp.int32, (_NCHUNK, S), 1)
    seg = (iota_col // _CHUNK == iota_row).astype(jnp.float32)
    cnt1 = jnp.dot(seg, oh1, preferred_element_type=jnp.float32)
    cnt2 = jnp.dot(seg, oh2, preferred_element_type=jnp.float32)

    ir = lax.broadcasted_iota(jnp.int32, (_NCHUNK, _NCHUNK), 0)
    ic = lax.broadcasted_iota(jnp.int32, (_NCHUNK, _NCHUNK), 1)
    stril = (ic < ir).astype(jnp.float32)
    excl1 = jnp.dot(stril, cnt1, preferred_element_type=jnp.float32)
    excl2 = jnp.dot(stril, cnt2, preferred_element_type=jnp.float32)

    total1 = jnp.sum(cnt1, axis=0, keepdims=True)
    cap1 = jnp.minimum(total1, jnp.float32(C))
    base2 = excl2 + cap1

    e1_ref[0] = e1
    e2_ref[0] = e2
    g1_ref[0] = mx1
    g2_ref[0] = mx2
    b1_ref[0] = excl1
    b2_ref[0] = base2

    d = jnp.float32(1.0 + 1e-6)
    proxy = (jnp.sum(raw, axis=0, keepdims=True) / S) / d
    dens = (total1 / S) / d
    aux_g = jnp.sum(proxy * dens)
    aux_ref[0] = jnp.full((8, 128), aux_g, dtype=jnp.float32)


def _sc_assign_kernel(e1_hbm, e2_hbm, g1_hbm, g2_hbm, b1_hbm, b2_hbm,
                      p1_hbm, p2_hbm, g1n_hbm, g2n_hbm,
                      e1v, e2v, g1v, g2v, b1v, b2v,
                      p1v, p2v, g1nv, g2nv, run1v, run2v, *, C):
    c = lax.axis_index("c")
    s = lax.axis_index("s")
    g = 2 * c + s // (_NCHUNK)
    chunk = s % _NCHUNK
    w = g * _NCHUNK + chunk
    base_tok = w * _CHUNK

    pltpu.sync_copy(e1_hbm.at[pl.ds(base_tok, _CHUNK)], e1v)
    pltpu.sync_copy(e2_hbm.at[pl.ds(base_tok, _CHUNK)], e2v)
    pltpu.sync_copy(g1_hbm.at[pl.ds(base_tok, _CHUNK)], g1v)
    pltpu.sync_copy(g2_hbm.at[pl.ds(base_tok, _CHUNK)], g2v)
    pltpu.sync_copy(b1_hbm.at[pl.ds(w * 16, 16)], b1v)
    pltpu.sync_copy(b2_hbm.at[pl.ds(w * 16, 16)], b2v)

    iotai = lax.iota(jnp.int32, 16)
    capv = jnp.full((16,), float(C), jnp.float32)
    zerov = jnp.zeros((16,), jnp.float32)
    onev = jnp.ones((16,), jnp.float32)

    def take16(x, idx):
        return x.at[idx].get(mode="promise_in_bounds")

    run1v[...] = b1v[...]
    run2v[...] = b2v[...]

    def vg_body(vg, carry):
        run1 = run1v[...]
        run2 = run2v[...]
        off = vg * 16
        ev1 = e1v[pl.ds(off, 16)].astype(jnp.int32)
        ev2 = e2v[pl.ds(off, 16)].astype(jnp.int32)
        gv1 = g1v[pl.ds(off, 16)]
        gv2 = g2v[pl.ds(off, 16)]
        pos1 = jnp.zeros((16,), jnp.float32)
        pos2 = jnp.zeros((16,), jnp.float32)
        for k in range(16):
            kvec = jnp.full((16,), k, jnp.int32)
            sel = iotai == kvec
            e1b = take16(ev1, kvec)
            pv1 = take16(run1, e1b)
            pos1 = jnp.where(sel, pv1, pos1)
            run1 = run1 + jnp.where(iotai == e1b, onev, zerov)
            e2b = take16(ev2, kvec)
            pv2 = take16(run2, e2b)
            pos2 = jnp.where(sel, pv2, pos2)
            run2 = run2 + jnp.where(iotai == e2b, onev, zerov)
        g1k = jnp.where(pos1 < capv, gv1, zerov)
        g2k = jnp.where(pos2 < capv, gv2, zerov)
        denom = g1k + g2k
        denom = jnp.where(denom > zerov, denom, onev)
        p1v[pl.ds(off, 16)] = pos1
        p2v[pl.ds(off, 16)] = pos2
        g1nv[pl.ds(off, 16)] = g1k / denom
        g2nv[pl.ds(off, 16)] = g2k / denom
        run1v[...] = run1
        run2v[...] = run2
        return carry

    lax.fori_loop(0, _CHUNK // 16, vg_body, 0)

    pltpu.sync_copy(p1v, p1_hbm.at[pl.ds(base_tok, _CHUNK)])
    pltpu.sync_copy(p2v, p2_hbm.at[pl.ds(base_tok, _CHUNK)])
    pltpu.sync_copy(g1nv, g1n_hbm.at[pl.ds(base_tok, _CHUNK)])
    pltpu.sync_copy(g2nv, g2n_hbm.at[pl.ds(base_tok, _CHUNK)])


def _construct_kernel(e1_ref, e2_ref, p1_ref, p2_ref, g1_ref, g2_ref,
                      comb_ref, disp_ref, *, SB, E, C):
    e1 = e1_ref[0]
    e2 = e2_ref[0]
    p1 = p1_ref[0]
    p2 = p2_ref[0]
    g1 = g1_ref[0]
    g2 = g2_ref[0]

    iota_e = lax.broadcasted_iota(jnp.int32, (SB, E), 1).astype(jnp.float32)
    iota_c = lax.broadcasted_iota(jnp.int32, (SB, C), 1).astype(jnp.float32)

    ohe1 = (iota_e == e1).astype(jnp.float32)
    ohe2 = (iota_e == e2).astype(jnp.float32)
    gc1 = g1 * (iota_c == p1).astype(jnp.float32)
    gc2 = g2 * (iota_c == p2).astype(jnp.float32)

    comb = ohe1[:, :, None] * gc1[:, None, :] + ohe2[:, :, None] * gc2[:, None, :]
    comb_ref[0] = comb
    disp_ref[0] = (comb != 0.0).astype(jnp.float32)


def kernel(inputs, gating_weight, total_token_num):
    G, S, M = inputs.shape
    E = gating_weight.shape[1]
    C = 256
    GS = G * S

    route = pl.pallas_call(
        functools.partial(_routing_kernel, S=S, E=E, C=C),
        grid=(G,),
        in_specs=[
            pl.BlockSpec((1, S, M), lambda g: (g, 0, 0)),
            pl.BlockSpec((M, E), lambda g: (0, 0)),
        ],
        out_specs=[pl.BlockSpec((1, S, 1), lambda g: (g, 0, 0))] * 4 + [
            pl.BlockSpec((1, _NCHUNK, E), lambda g: (g, 0, 0)),
            pl.BlockSpec((1, _NCHUNK, E), lambda g: (g, 0, 0)),
            pl.BlockSpec((1, 8, 128), lambda g: (g, 0, 0)),
        ],
        out_shape=[jax.ShapeDtypeStruct((G, S, 1), jnp.float32)] * 4 + [
            jax.ShapeDtypeStruct((G, _NCHUNK, E), jnp.float32),
            jax.ShapeDtypeStruct((G, _NCHUNK, E), jnp.float32),
            jax.ShapeDtypeStruct((G, 8, 128), jnp.float32),
        ],
    )
    e1, e2, g1, g2, b1, b2, auxp = route(inputs, gating_weight)

    mesh = plsc.VectorSubcoreMesh(core_axis_name="c", subcore_axis_name="s")
    assign = pl.kernel(
        functools.partial(_sc_assign_kernel, C=C),
        out_type=[jax.ShapeDtypeStruct((GS,), jnp.float32)] * 4,
        mesh=mesh,
        scratch_types=(
            [pltpu.VMEM((_CHUNK,), jnp.float32)] * 4
            + [pltpu.VMEM((16,), jnp.float32)] * 2
            + [pltpu.VMEM((_CHUNK,), jnp.float32)] * 4
            + [pltpu.VMEM((16,), jnp.float32)] * 2
        ),
    )
    p1, p2, g1n, g2n = assign(
        e1.reshape(GS), e2.reshape(GS), g1.reshape(GS), g2.reshape(GS),
        b1.reshape(G * _NCHUNK * E), b2.reshape(G * _NCHUNK * E))

    SB = 256
    NSB = S // SB
    tok_spec = pl.BlockSpec((1, SB, 1), lambda g, sb: (g, sb, 0))
    construct = pl.pallas_call(
        functools.partial(_construct_kernel, SB=SB, E=E, C=C),
        grid=(G, NSB),
        in_specs=[tok_spec] * 6,
        out_specs=[
            pl.BlockSpec((1, SB, E, C), lambda g, sb: (g, sb, 0, 0)),
            pl.BlockSpec((1, SB, E, C), lambda g, sb: (g, sb, 0, 0)),
        ],
        out_shape=[
            jax.ShapeDtypeStruct((G, S, E, C), jnp.float32),
            jax.ShapeDtypeStruct((G, S, E, C), jnp.float32),
        ],
    )
    shp = (G, S, 1)
    combine_tensor, dispatch_mask = construct(
        e1, e2, p1.reshape(shp), p2.reshape(shp),
        g1n.reshape(shp), g2n.reshape(shp))

    aux_loss = jnp.sum(auxp[:, 0, 0]) * jnp.float32(E) / jnp.float32(G)
    return combine_tensor, dispatch_mask, aux_loss


# speedup vs baseline: 1.2794x; 1.2794x over previous
import functools

import jax
import jax.numpy as jnp
from jax import lax
from jax.experimental import pallas as pl
from jax.experimental.pallas import tpu as pltpu
from jax.experimental.pallas import tpu_sc as plsc


def _routing_kernel(x_ref, w_ref, e1_ref, e2_ref, p1_ref, p2_ref,
                    g1_ref, g2_ref, st_ref, *, S, E, C):
    x = x_ref[0]
    w = w_ref[...]
    logits = jnp.dot(x, w, preferred_element_type=jnp.float32)

    m = jnp.max(logits, axis=-1, keepdims=True)
    ex = jnp.exp(logits - m)
    raw = ex / jnp.sum(ex, axis=-1, keepdims=True)

    iota_e = lax.broadcasted_iota(jnp.int32, (S, E), 1).astype(jnp.float32)

    mx1 = jnp.max(raw, axis=-1, keepdims=True)
    e1 = jnp.min(jnp.where(raw == mx1, iota_e, jnp.float32(E)),
                 axis=-1, keepdims=True)
    oh1 = (iota_e == e1).astype(jnp.float32)

    raw2 = raw * (1.0 - oh1)
    mx2 = jnp.max(raw2, axis=-1, keepdims=True)
    e2 = jnp.min(jnp.where(raw2 == mx2, iota_e, jnp.float32(E)),
                 axis=-1, keepdims=True)
    oh2 = (iota_e == e2).astype(jnp.float32)

    def _cumsum0(x):
        k = 1
        while k < x.shape[0]:
            shifted = jnp.concatenate(
                [jnp.zeros((k, x.shape[1]), x.dtype), x[:-k]], axis=0)
            x = x + shifted
            k *= 2
        return x

    cs1 = _cumsum0(oh1)
    cs2 = _cumsum0(oh2)
    pos1 = jnp.sum((cs1 - oh1) * oh1, axis=-1, keepdims=True)
    total1 = jnp.sum(oh1, axis=0, keepdims=True)
    cap1 = jnp.minimum(total1, jnp.float32(C))
    pos2 = (jnp.sum((cs2 - oh2) * oh2, axis=-1, keepdims=True)
            + jnp.sum(oh2 * cap1, axis=-1, keepdims=True))

    keep1 = (pos1 < C).astype(jnp.float32)
    keep2 = (pos2 < C).astype(jnp.float32)
    g1 = mx1 * keep1
    g2 = mx2 * keep2
    denom = g1 + g2
    denom = jnp.where(denom > 0, denom, 1.0)

    e1_ref[0] = e1
    e2_ref[0] = e2
    p1_ref[0] = pos1
    p2_ref[0] = pos2
    g1_ref[0] = g1 / denom
    g2_ref[0] = g2 / denom

    sum_raw = jnp.sum(raw, axis=0, keepdims=True)
    st_ref[0] = jnp.concatenate(
        [sum_raw, total1, jnp.zeros((6, E), jnp.float32)], axis=0)


def _sc_aux_kernel(st_hbm, aux_hbm, rowv, accv, *, S, E, G):
    c = lax.axis_index("c")
    s = lax.axis_index("s")

    @pl.when(jnp.logical_and(c == 0, s == 0))
    def _():
        dv = jnp.full((16,), float(S) * (1.0 + 1e-6), jnp.float32)
        accv[...] = jnp.zeros((16,), jnp.float32)
        for g in range(G):
            pltpu.sync_copy(st_hbm.at[pl.ds(g * 8 * E, 16)], rowv)
            proxy = rowv[...] / dv
            pltpu.sync_copy(st_hbm.at[pl.ds((g * 8 + 1) * E, 16)], rowv)
            dens = rowv[...] / dv
            accv[...] = accv[...] + proxy * dens
        pltpu.sync_copy(accv, aux_hbm)


def _construct_kernel(e1_ref, e2_ref, p1_ref, p2_ref, g1_ref, g2_ref,
                      comb_ref, disp_ref, *, SB, E, C):
    e1 = e1_ref[0]
    e2 = e2_ref[0]
    p1 = p1_ref[0]
    p2 = p2_ref[0]
    g1 = g1_ref[0]
    g2 = g2_ref[0]

    iota_e = lax.broadcasted_iota(jnp.int32, (SB, E), 1).astype(jnp.float32)
    iota_c = lax.broadcasted_iota(jnp.int32, (SB, C), 1).astype(jnp.float32)

    ohe1 = (iota_e == e1).astype(jnp.float32)
    ohe2 = (iota_e == e2).astype(jnp.float32)
    gc1 = g1 * (iota_c == p1).astype(jnp.float32)
    gc2 = g2 * (iota_c == p2).astype(jnp.float32)

    comb = ohe1[:, :, None] * gc1[:, None, :] + ohe2[:, :, None] * gc2[:, None, :]
    comb_ref[0] = comb
    disp_ref[0] = (comb != 0.0).astype(jnp.float32)


def kernel(inputs, gating_weight, total_token_num):
    G, S, M = inputs.shape
    E = gating_weight.shape[1]
    C = 256

    route = pl.pallas_call(
        functools.partial(_routing_kernel, S=S, E=E, C=C),
        grid=(G,),
        in_specs=[
            pl.BlockSpec((1, S, M), lambda g: (g, 0, 0)),
            pl.BlockSpec((M, E), lambda g: (0, 0)),
        ],
        out_specs=[pl.BlockSpec((1, S, 1), lambda g: (g, 0, 0))] * 6 + [
            pl.BlockSpec((1, 8, E), lambda g: (g, 0, 0)),
        ],
        out_shape=[jax.ShapeDtypeStruct((G, S, 1), jnp.float32)] * 6 + [
            jax.ShapeDtypeStruct((G, 8, E), jnp.float32),
        ],
    )
    e1, e2, p1, p2, g1, g2, stats = route(inputs, gating_weight)

    mesh = plsc.VectorSubcoreMesh(core_axis_name="c", subcore_axis_name="s")
    aux_partial = pl.kernel(
        functools.partial(_sc_aux_kernel, S=S, E=E, G=G),
        out_type=jax.ShapeDtypeStruct((16,), jnp.float32),
        mesh=mesh,
        scratch_types=[
            pltpu.VMEM((16,), jnp.float32),
            pltpu.VMEM((16,), jnp.float32),
        ],
    )(stats.reshape(G * 8 * E))

    SB = 256
    NSB = S // SB
    tok_spec = pl.BlockSpec((1, SB, 1), lambda g, sb: (g, sb, 0))
    construct = pl.pallas_call(
        functools.partial(_construct_kernel, SB=SB, E=E, C=C),
        grid=(G, NSB),
        in_specs=[tok_spec] * 6,
        out_specs=[
            pl.BlockSpec((1, SB, E, C), lambda g, sb: (g, sb, 0, 0)),
            pl.BlockSpec((1, SB, E, C), lambda g, sb: (g, sb, 0, 0)),
        ],
        out_shape=[
            jax.ShapeDtypeStruct((G, S, E, C), jnp.float32),
            jax.ShapeDtypeStruct((G, S, E, C), jnp.float32),
        ],
    )
    combine_tensor, dispatch_mask = construct(e1, e2, p1, p2, g1, g2)

    aux_loss = jnp.sum(aux_partial) * jnp.float32(E) / jnp.float32(G)
    return combine_tensor, dispatch_mask, aux_loss


# speedup vs baseline: 1.2951x; 1.0122x over previous
import functools

import jax
import jax.numpy as jnp
from jax import lax
from jax.experimental import pallas as pl
from jax.experimental.pallas import tpu as pltpu
from jax.experimental.pallas import tpu_sc as plsc


def _routing_kernel(x_ref, w_ref, e1_ref, e2_ref, p1_ref, p2_ref,
                    g1_ref, g2_ref, st_ref, *, S, E, C):
    x = x_ref[0]
    w = w_ref[...]
    logits = jnp.dot(x, w, preferred_element_type=jnp.float32)

    m = jnp.max(logits, axis=-1, keepdims=True)
    ex = jnp.exp(logits - m)
    raw = ex / jnp.sum(ex, axis=-1, keepdims=True)

    iota_e = lax.broadcasted_iota(jnp.int32, (S, E), 1).astype(jnp.float32)

    mx1 = jnp.max(raw, axis=-1, keepdims=True)
    e1 = jnp.min(jnp.where(raw == mx1, iota_e, jnp.float32(E)),
                 axis=-1, keepdims=True)
    oh1 = (iota_e == e1).astype(jnp.float32)

    raw2 = raw * (1.0 - oh1)
    mx2 = jnp.max(raw2, axis=-1, keepdims=True)
    e2 = jnp.min(jnp.where(raw2 == mx2, iota_e, jnp.float32(E)),
                 axis=-1, keepdims=True)
    oh2 = (iota_e == e2).astype(jnp.float32)

    def _cumsum0(x):
        k = 1
        while k < x.shape[0]:
            shifted = jnp.concatenate(
                [jnp.zeros((k, x.shape[1]), x.dtype), x[:-k]], axis=0)
            x = x + shifted
            k *= 2
        return x

    cs1 = _cumsum0(oh1)
    cs2 = _cumsum0(oh2)
    pos1 = jnp.sum((cs1 - oh1) * oh1, axis=-1, keepdims=True)
    total1 = jnp.sum(oh1, axis=0, keepdims=True)
    cap1 = jnp.minimum(total1, jnp.float32(C))
    pos2 = (jnp.sum((cs2 - oh2) * oh2, axis=-1, keepdims=True)
            + jnp.sum(oh2 * cap1, axis=-1, keepdims=True))

    keep1 = (pos1 < C).astype(jnp.float32)
    keep2 = (pos2 < C).astype(jnp.float32)
    g1 = mx1 * keep1
    g2 = mx2 * keep2
    denom = g1 + g2
    denom = jnp.where(denom > 0, denom, 1.0)

    e1_ref[0] = e1
    e2_ref[0] = e2
    p1_ref[0] = pos1
    p2_ref[0] = pos2
    g1_ref[0] = g1 / denom
    g2_ref[0] = g2 / denom

    sum_raw = jnp.sum(raw, axis=0, keepdims=True)
    st_ref[0] = jnp.concatenate(
        [sum_raw, total1, jnp.zeros((6, E), jnp.float32)], axis=0)


def _sc_aux_kernel(st_hbm, aux_hbm, rowv, accv, *, S, E, G):
    c = lax.axis_index("c")
    s = lax.axis_index("s")

    @pl.when(jnp.logical_and(c == 0, s == 0))
    def _():
        dv = jnp.full((16,), float(S) * (1.0 + 1e-6), jnp.float32)
        accv[...] = jnp.zeros((16,), jnp.float32)
        for g in range(G):
            pltpu.sync_copy(st_hbm.at[pl.ds(g * 8 * E, 16)], rowv)
            proxy = rowv[...] / dv
            pltpu.sync_copy(st_hbm.at[pl.ds((g * 8 + 1) * E, 16)], rowv)
            dens = rowv[...] / dv
            accv[...] = accv[...] + proxy * dens
        pltpu.sync_copy(accv, aux_hbm)


def _construct_kernel(e1_ref, e2_ref, p1_ref, p2_ref, g1_ref, g2_ref,
                      comb_ref, disp_ref, *, SB, E, C):
    e1 = e1_ref[0]
    e2 = e2_ref[0]
    p1 = p1_ref[0]
    p2 = p2_ref[0]
    g1 = g1_ref[0]
    g2 = g2_ref[0]

    iota_e = lax.broadcasted_iota(jnp.int32, (SB, E), 1).astype(jnp.float32)
    iota_c = lax.broadcasted_iota(jnp.int32, (SB, C), 1).astype(jnp.float32)

    ohe1 = (iota_e == e1).astype(jnp.float32)
    ohe2 = (iota_e == e2).astype(jnp.float32)
    gc1 = g1 * (iota_c == p1).astype(jnp.float32)
    gc2 = g2 * (iota_c == p2).astype(jnp.float32)

    comb = ohe1[:, :, None] * gc1[:, None, :] + ohe2[:, :, None] * gc2[:, None, :]
    comb_ref[0] = comb
    disp_ref[0] = (comb != 0.0).astype(jnp.float32)


def kernel(inputs, gating_weight, total_token_num):
    G, S, M = inputs.shape
    E = gating_weight.shape[1]
    C = 256

    route = pl.pallas_call(
        functools.partial(_routing_kernel, S=S, E=E, C=C),
        grid=(G,),
        in_specs=[
            pl.BlockSpec((1, S, M), lambda g: (g, 0, 0)),
            pl.BlockSpec((M, E), lambda g: (0, 0)),
        ],
        out_specs=[pl.BlockSpec((1, S, 1), lambda g: (g, 0, 0))] * 6 + [
            pl.BlockSpec((1, 8, E), lambda g: (g, 0, 0)),
        ],
        out_shape=[jax.ShapeDtypeStruct((G, S, 1), jnp.float32)] * 6 + [
            jax.ShapeDtypeStruct((G, 8, E), jnp.float32),
        ],
    )
    e1, e2, p1, p2, g1, g2, stats = route(inputs, gating_weight)

    mesh = plsc.VectorSubcoreMesh(core_axis_name="c", subcore_axis_name="s", num_cores=1)
    aux_partial = pl.kernel(
        functools.partial(_sc_aux_kernel, S=S, E=E, G=G),
        out_type=jax.ShapeDtypeStruct((16,), jnp.float32),
        mesh=mesh,
        scratch_types=[
            pltpu.VMEM((16,), jnp.float32),
            pltpu.VMEM((16,), jnp.float32),
        ],
    )(stats.reshape(G * 8 * E))

    SB = 256
    NSB = S // SB
    tok_spec = pl.BlockSpec((1, SB, 1), lambda g, sb: (g, sb, 0))
    construct = pl.pallas_call(
        functools.partial(_construct_kernel, SB=SB, E=E, C=C),
        grid=(G, NSB),
        in_specs=[tok_spec] * 6,
        out_specs=[
            pl.BlockSpec((1, SB, E, C), lambda g, sb: (g, sb, 0, 0)),
            pl.BlockSpec((1, SB, E, C), lambda g, sb: (g, sb, 0, 0)),
        ],
        out_shape=[
            jax.ShapeDtypeStruct((G, S, E, C), jnp.float32),
            jax.ShapeDtypeStruct((G, S, E, C), jnp.float32),
        ],
    )
    combine_tensor, dispatch_mask = construct(e1, e2, p1, p2, g1, g2)

    aux_loss = jnp.sum(aux_partial) * jnp.float32(E) / jnp.float32(G)
    return combine_tensor, dispatch_mask, aux_loss


# speedup vs baseline: 1.2980x; 1.0023x over previous
import functools

import jax
import jax.numpy as jnp
from jax import lax
from jax.experimental import pallas as pl
from jax.experimental.pallas import tpu as pltpu
from jax.experimental.pallas import tpu_sc as plsc


def _routing_kernel(x_ref, w_ref, e1_ref, e2_ref, p1_ref, p2_ref,
                    g1_ref, g2_ref, st_ref, *, S, E, C):
    x = x_ref[0]
    w = w_ref[...]
    logits = jnp.dot(x, w, preferred_element_type=jnp.float32)

    m = jnp.max(logits, axis=-1, keepdims=True)
    ex = jnp.exp(logits - m)
    raw = ex / jnp.sum(ex, axis=-1, keepdims=True)

    iota_e = lax.broadcasted_iota(jnp.int32, (S, E), 1).astype(jnp.float32)

    mx1 = jnp.max(raw, axis=-1, keepdims=True)
    e1 = jnp.min(jnp.where(raw == mx1, iota_e, jnp.float32(E)),
                 axis=-1, keepdims=True)
    oh1 = (iota_e == e1).astype(jnp.float32)

    raw2 = raw * (1.0 - oh1)
    mx2 = jnp.max(raw2, axis=-1, keepdims=True)
    e2 = jnp.min(jnp.where(raw2 == mx2, iota_e, jnp.float32(E)),
                 axis=-1, keepdims=True)
    oh2 = (iota_e == e2).astype(jnp.float32)

    def _cumsum0(x):
        k = 1
        while k < x.shape[0]:
            shifted = jnp.concatenate(
                [jnp.zeros((k, x.shape[1]), x.dtype), x[:-k]], axis=0)
            x = x + shifted
            k *= 2
        return x

    cs1 = _cumsum0(oh1)
    cs2 = _cumsum0(oh2)
    pos1 = jnp.sum((cs1 - oh1) * oh1, axis=-1, keepdims=True)
    total1 = jnp.sum(oh1, axis=0, keepdims=True)
    cap1 = jnp.minimum(total1, jnp.float32(C))
    pos2 = (jnp.sum((cs2 - oh2) * oh2, axis=-1, keepdims=True)
            + jnp.sum(oh2 * cap1, axis=-1, keepdims=True))

    keep1 = (pos1 < C).astype(jnp.float32)
    keep2 = (pos2 < C).astype(jnp.float32)
    g1 = mx1 * keep1
    g2 = mx2 * keep2
    denom = g1 + g2
    denom = jnp.where(denom > 0, denom, 1.0)

    e1_ref[0] = e1
    e2_ref[0] = e2
    p1_ref[0] = pos1
    p2_ref[0] = pos2
    g1_ref[0] = g1 / denom
    g2_ref[0] = g2 / denom

    sum_raw = jnp.sum(raw, axis=0, keepdims=True)
    st_ref[0] = jnp.concatenate(
        [sum_raw, total1, jnp.zeros((6, E), jnp.float32)], axis=0)


def _sc_aux_kernel(st_hbm, aux_hbm, rowv, accv, *, S, E, G):
    c = lax.axis_index("c")
    s = lax.axis_index("s")

    @pl.when(jnp.logical_and(c == 0, s == 0))
    def _():
        dv = jnp.full((16,), float(S) * (1.0 + 1e-6), jnp.float32)
        accv[...] = jnp.zeros((16,), jnp.float32)
        for g in range(G):
            pltpu.sync_copy(st_hbm.at[pl.ds(g * 8 * E, 16)], rowv)
            proxy = rowv[...] / dv
            pltpu.sync_copy(st_hbm.at[pl.ds((g * 8 + 1) * E, 16)], rowv)
            dens = rowv[...] / dv
            accv[...] = accv[...] + proxy * dens
        pltpu.sync_copy(accv, aux_hbm)


def _construct_kernel(e1_ref, e2_ref, p1_ref, p2_ref, g1_ref, g2_ref,
                      comb_ref, disp_ref, *, SB, E, C):
    e1 = e1_ref[0]
    e2 = e2_ref[0]
    p1 = p1_ref[0]
    p2 = p2_ref[0]
    g1 = g1_ref[0]
    g2 = g2_ref[0]

    iota_e = lax.broadcasted_iota(jnp.int32, (SB, E), 1).astype(jnp.float32)
    iota_c = lax.broadcasted_iota(jnp.int32, (SB, C), 1).astype(jnp.float32)

    ohe1 = (iota_e == e1).astype(jnp.float32)
    ohe2 = (iota_e == e2).astype(jnp.float32)
    gc1 = g1 * (iota_c == p1).astype(jnp.float32)
    gc2 = g2 * (iota_c == p2).astype(jnp.float32)

    comb = ohe1[:, :, None] * gc1[:, None, :] + ohe2[:, :, None] * gc2[:, None, :]
    comb_ref[0] = comb
    disp_ref[0] = (comb != 0.0).astype(jnp.float32)


def kernel(inputs, gating_weight, total_token_num):
    G, S, M = inputs.shape
    E = gating_weight.shape[1]
    C = 256

    route = pl.pallas_call(
        functools.partial(_routing_kernel, S=S, E=E, C=C),
        grid=(G,),
        in_specs=[
            pl.BlockSpec((1, S, M), lambda g: (g, 0, 0)),
            pl.BlockSpec((M, E), lambda g: (0, 0)),
        ],
        out_specs=[pl.BlockSpec((1, S, 1), lambda g: (g, 0, 0))] * 6 + [
            pl.BlockSpec((1, 8, E), lambda g: (g, 0, 0)),
        ],
        out_shape=[jax.ShapeDtypeStruct((G, S, 1), jnp.float32)] * 6 + [
            jax.ShapeDtypeStruct((G, 8, E), jnp.float32),
        ],
    )
    e1, e2, p1, p2, g1, g2, stats = route(inputs, gating_weight)

    mesh = plsc.VectorSubcoreMesh(core_axis_name="c", subcore_axis_name="s", num_cores=1, num_subcores=1)
    aux_partial = pl.kernel(
        functools.partial(_sc_aux_kernel, S=S, E=E, G=G),
        out_type=jax.ShapeDtypeStruct((16,), jnp.float32),
        mesh=mesh,
        scratch_types=[
            pltpu.VMEM((16,), jnp.float32),
            pltpu.VMEM((16,), jnp.float32),
        ],
    )(stats.reshape(G * 8 * E))

    SB = 256
    NSB = S // SB
    tok_spec = pl.BlockSpec((1, SB, 1), lambda g, sb: (g, sb, 0))
    construct = pl.pallas_call(
        functools.partial(_construct_kernel, SB=SB, E=E, C=C),
        grid=(G, NSB),
        in_specs=[tok_spec] * 6,
        out_specs=[
            pl.BlockSpec((1, SB, E, C), lambda g, sb: (g, sb, 0, 0)),
            pl.BlockSpec((1, SB, E, C), lambda g, sb: (g, sb, 0, 0)),
        ],
        out_shape=[
            jax.ShapeDtypeStruct((G, S, E, C), jnp.float32),
            jax.ShapeDtypeStruct((G, S, E, C), jnp.float32),
        ],
    )
    combine_tensor, dispatch_mask = construct(e1, e2, p1, p2, g1, g2)

    aux_loss = jnp.sum(aux_partial) * jnp.float32(E) / jnp.float32(G)
    return combine_tensor, dispatch_mask, aux_loss


# speedup vs baseline: 1.3006x; 1.0020x over previous
import functools

import jax
import jax.numpy as jnp
from jax import lax
from jax.experimental import pallas as pl
from jax.experimental.pallas import tpu as pltpu
from jax.experimental.pallas import tpu_sc as plsc


def _routing_kernel(x_ref, w_ref, e1_ref, e2_ref, p1_ref, p2_ref,
                    g1_ref, g2_ref, st_ref, *, S, E, C):
    x = x_ref[0]
    w = w_ref[...]
    logits = jnp.dot(x, w, preferred_element_type=jnp.float32)

    m = jnp.max(logits, axis=-1, keepdims=True)
    ex = jnp.exp(logits - m)
    raw = ex / jnp.sum(ex, axis=-1, keepdims=True)

    iota_e = lax.broadcasted_iota(jnp.int32, (S, E), 1).astype(jnp.float32)

    mx1 = jnp.max(raw, axis=-1, keepdims=True)
    e1 = jnp.min(jnp.where(raw == mx1, iota_e, jnp.float32(E)),
                 axis=-1, keepdims=True)
    oh1 = (iota_e == e1).astype(jnp.float32)

    raw2 = raw * (1.0 - oh1)
    mx2 = jnp.max(raw2, axis=-1, keepdims=True)
    e2 = jnp.min(jnp.where(raw2 == mx2, iota_e, jnp.float32(E)),
                 axis=-1, keepdims=True)
    oh2 = (iota_e == e2).astype(jnp.float32)

    def _cumsum0(x):
        k = 1
        while k < x.shape[0]:
            shifted = jnp.concatenate(
                [jnp.zeros((k, x.shape[1]), x.dtype), x[:-k]], axis=0)
            x = x + shifted
            k *= 2
        return x

    cs1 = _cumsum0(oh1)
    cs2 = _cumsum0(oh2)
    pos1 = jnp.sum((cs1 - oh1) * oh1, axis=-1, keepdims=True)
    total1 = jnp.sum(oh1, axis=0, keepdims=True)
    cap1 = jnp.minimum(total1, jnp.float32(C))
    pos2 = (jnp.sum((cs2 - oh2) * oh2, axis=-1, keepdims=True)
            + jnp.sum(oh2 * cap1, axis=-1, keepdims=True))

    keep1 = (pos1 < C).astype(jnp.float32)
    keep2 = (pos2 < C).astype(jnp.float32)
    g1 = mx1 * keep1
    g2 = mx2 * keep2
    denom = g1 + g2
    denom = jnp.where(denom > 0, denom, 1.0)

    e1_ref[0] = e1
    e2_ref[0] = e2
    p1_ref[0] = pos1
    p2_ref[0] = pos2
    g1_ref[0] = g1 / denom
    g2_ref[0] = g2 / denom

    sum_raw = jnp.sum(raw, axis=0, keepdims=True)
    st_ref[0] = jnp.concatenate(
        [sum_raw, total1, jnp.zeros((6, E), jnp.float32)], axis=0)


def _sc_aux_kernel(st_hbm, aux_hbm, rowv, accv, *, S, E, G):
    c = lax.axis_index("c")
    s = lax.axis_index("s")

    @pl.when(jnp.logical_and(c == 0, s == 0))
    def _():
        dv = jnp.full((16,), float(S) * (1.0 + 1e-6), jnp.float32)
        accv[...] = jnp.zeros((16,), jnp.float32)
        for g in range(G):
            pltpu.sync_copy(st_hbm.at[pl.ds(g * 8 * E, 16)], rowv)
            proxy = rowv[...] / dv
            pltpu.sync_copy(st_hbm.at[pl.ds((g * 8 + 1) * E, 16)], rowv)
            dens = rowv[...] / dv
            accv[...] = accv[...] + proxy * dens
        pltpu.sync_copy(accv, aux_hbm)


def _construct_kernel(e1_ref, e2_ref, p1_ref, p2_ref, g1_ref, g2_ref,
                      comb_ref, disp_ref, *, SB, E, C):
    e1 = e1_ref[0]
    e2 = e2_ref[0]
    p1 = p1_ref[0]
    p2 = p2_ref[0]
    g1 = g1_ref[0]
    g2 = g2_ref[0]

    iota_e = lax.broadcasted_iota(jnp.int32, (SB, E), 1).astype(jnp.float32)
    iota_c = lax.broadcasted_iota(jnp.int32, (SB, C), 1).astype(jnp.float32)

    ohe1 = (iota_e == e1).astype(jnp.float32)
    ohe2 = (iota_e == e2).astype(jnp.float32)
    gc1 = g1 * (iota_c == p1).astype(jnp.float32)
    gc2 = g2 * (iota_c == p2).astype(jnp.float32)

    comb = ohe1[:, :, None] * gc1[:, None, :] + ohe2[:, :, None] * gc2[:, None, :]
    comb_ref[0] = comb
    disp_ref[0] = (comb != 0.0).astype(jnp.float32)


def kernel(inputs, gating_weight, total_token_num):
    G, S, M = inputs.shape
    E = gating_weight.shape[1]
    C = 256

    route = pl.pallas_call(
        functools.partial(_routing_kernel, S=S, E=E, C=C),
        grid=(G,),
        in_specs=[
            pl.BlockSpec((1, S, M), lambda g: (g, 0, 0)),
            pl.BlockSpec((M, E), lambda g: (0, 0)),
        ],
        out_specs=[pl.BlockSpec((1, S, 1), lambda g: (g, 0, 0))] * 6 + [
            pl.BlockSpec((1, 8, E), lambda g: (g, 0, 0)),
        ],
        out_shape=[jax.ShapeDtypeStruct((G, S, 1), jnp.float32)] * 6 + [
            jax.ShapeDtypeStruct((G, 8, E), jnp.float32),
        ],
    )
    e1, e2, p1, p2, g1, g2, stats = route(inputs, gating_weight)

    mesh = plsc.VectorSubcoreMesh(core_axis_name="c", subcore_axis_name="s", num_cores=1, num_subcores=1)
    aux_partial = pl.kernel(
        functools.partial(_sc_aux_kernel, S=S, E=E, G=G),
        out_type=jax.ShapeDtypeStruct((16,), jnp.float32),
        mesh=mesh,
        scratch_types=[
            pltpu.VMEM((16,), jnp.float32),
            pltpu.VMEM((16,), jnp.float32),
        ],
    )(stats.reshape(G * 8 * E))

    SB = 512
    NSB = S // SB
    tok_spec = pl.BlockSpec((1, SB, 1), lambda g, sb: (g, sb, 0))
    construct = pl.pallas_call(
        functools.partial(_construct_kernel, SB=SB, E=E, C=C),
        grid=(G, NSB),
        in_specs=[tok_spec] * 6,
        out_specs=[
            pl.BlockSpec((1, SB, E, C), lambda g, sb: (g, sb, 0, 0)),
            pl.BlockSpec((1, SB, E, C), lambda g, sb: (g, sb, 0, 0)),
        ],
        out_shape=[
            jax.ShapeDtypeStruct((G, S, E, C), jnp.float32),
            jax.ShapeDtypeStruct((G, S, E, C), jnp.float32),
        ],
    )
    combine_tensor, dispatch_mask = construct(e1, e2, p1, p2, g1, g2)

    aux_loss = jnp.sum(aux_partial) * jnp.float32(E) / jnp.float32(G)
    return combine_tensor, dispatch_mask, aux_loss
